# baseline (speedup 1.0000x reference)
import functools

import jax
import jax.numpy as jnp
from jax import lax
from jax.experimental import pallas as pl
from jax.experimental.pallas import tpu as pltpu
from jax.experimental.pallas import tpu_sc as plsc

B = 16384
D = 32
KP1 = 21
NC, NS = 2, 16
NW = NC * NS
PER_W = B // NW
CHUNK = 128
NCHUNK = PER_W // CHUNK
C21 = CHUNK * KP1


def _sc_scores(center, cidx2d, in_embed, out_embed):
    mesh = plsc.VectorSubcoreMesh(
        core_axis_name="c", subcore_axis_name="s",
        num_cores=NC, num_subcores=NS)

    @functools.partial(
        pl.kernel,
        out_type=jax.ShapeDtypeStruct((B * KP1,), jnp.float32),
        mesh=mesh,
        compiler_params=pltpu.CompilerParams(
            use_tc_tiling_on_sc=False, needs_layout_passes=False),
        scratch_types=[
            pltpu.VMEM((CHUNK,), jnp.int32),
            pltpu.VMEM((C21,), jnp.int32),
            pltpu.VMEM((CHUNK, D), jnp.float32),
            pltpu.VMEM((C21, D), jnp.float32),
            pltpu.VMEM((C21,), jnp.float32),
            pltpu.SemaphoreType.DMA,
        ],
    )
    def k(center_hbm, cidx_hbm, inemb_hbm, outemb_hbm, out_hbm,
          cen_v, cidx_v, crows_v, orows_v, sc_v, sem):
        wid = lax.axis_index("s") * NC + lax.axis_index("c")
        iota = lax.iota(jnp.int32, 16)

        def chunk_body(c, _):
            base_b = pl.multiple_of(wid * PER_W + c * CHUNK, CHUNK)
            base21 = pl.multiple_of(base_b * KP1, C21)
            pltpu.sync_copy(center_hbm.at[pl.ds(base_b, CHUNK)], cen_v)
            pltpu.sync_copy(cidx_hbm.at[pl.ds(base21, C21)], cidx_v)
            cps = [pltpu.async_copy(inemb_hbm.at[cen_v], crows_v, sem)]
            for r in range(KP1):
                cps.append(pltpu.async_copy(
                    outemb_hbm.at[cidx_v.at[pl.ds(r * CHUNK, CHUNK)]],
                    orows_v.at[pl.ds(r * CHUNK, CHUNK)], sem))
            for cp in cps:
                cp.wait()

            def g_body(g, _):
                r16 = g * 16 + iota
                ccols = [
                    plsc.load_gather(
                        crows_v, [r16, jnp.full((16,), d, jnp.int32)])
                    for d in range(D)
                ]
                for j in range(KP1):
                    rj = g * (16 * KP1) + j + iota * KP1
                    s = ccols[0] * plsc.load_gather(
                        orows_v, [rj, jnp.full((16,), 0, jnp.int32)])
                    for d in range(1, D):
                        s = s + ccols[d] * plsc.load_gather(
                            orows_v, [rj, jnp.full((16,), d, jnp.int32)])
                    if j:
                        s = -s
                    plsc.store_scatter(sc_v, [rj], s)
                return 0

            lax.fori_loop(0, CHUNK // 16, g_body, 0)
            pltpu.sync_copy(sc_v, out_hbm.at[pl.ds(base21, C21)])
            return 0

        lax.fori_loop(0, NCHUNK, chunk_body, 0)

    return k(center, cidx2d, in_embed, out_embed)


def _tc_loss(scores):
    x2 = scores.reshape(B * KP1 // 128, 128)

    def body(x_ref, o_ref):
        x = x_ref[...]
        ls = jnp.minimum(x, 0.0) - jnp.log1p(jnp.exp(-jnp.abs(x)))
        o_ref[0, 0] = -jnp.sum(ls) * (1.0 / B)

    out = pl.pallas_call(
        body,
        out_shape=jax.ShapeDtypeStruct((1, 1), jnp.float32),
        out_specs=pl.BlockSpec(memory_space=pltpu.SMEM),
    )(x2)
    return out[0, 0]


def kernel(center, context, negatives, in_embed, out_embed):
    cidx = jnp.concatenate([context[:, None], negatives], axis=1)
    scores = _sc_scores(center, cidx.reshape(B * KP1), in_embed, out_embed)
    return _tc_loss(scores)
